# baseline (speedup 1.0000x reference)
import functools

import jax
import jax.numpy as jnp
from jax import lax
from jax.experimental import pallas as pl
from jax.experimental.pallas import tpu as pltpu
from jax.experimental.pallas import tpu_sc as plsc

N = 4096
D = 64
K = 512
NCLS = 10
CSL = 16

ROWS = 1024
GRID = N // ROWS

NTILES = 16
KPT = K // NTILES
BINS = KPT * CSL
NVEC = N // 16


def _tc_body(x_ref, c_ref, y_ref, loss_ref, key_ref):
    i = pl.program_id(0)
    xb = x_ref[...]
    cb = c_ref[...]
    xc = lax.dot_general(
        xb, cb, (((1,), (1,)), ((), ())),
        preferred_element_type=jnp.float32,
        precision=lax.Precision.HIGHEST,
    )
    xn = jnp.sum(xb * xb, axis=1)
    cn = jnp.sum(cb * cb, axis=1)
    dist = xn[:, None] + cn[None, :] - 2.0 * xc
    minv = jnp.min(dist, axis=1)
    col = lax.broadcasted_iota(jnp.int32, (ROWS, K), 1)
    yp = jnp.min(jnp.where(dist == minv[:, None], col, K), axis=1)
    key_ref[0, :, 0] = yp * CSL + y_ref[0, :, 0]

    @pl.when(i == 0)
    def _():
        loss_ref[0, 0] = 0.0

    loss_ref[0, 0] += jnp.sum(minv)


def _sc_hist(key):
    mesh = plsc.VectorSubcoreMesh(core_axis_name="c", subcore_axis_name="s")

    @functools.partial(
        pl.kernel,
        out_type=jax.ShapeDtypeStruct((16,), jnp.float32),
        mesh=mesh,
        compiler_params=pltpu.CompilerParams(needs_layout_passes=False),
        scratch_types=[
            pltpu.VMEM((N,), jnp.int32),
            pltpu.VMEM((BINS,), jnp.int32),
            pltpu.VMEM((16,), jnp.float32),
            pltpu.SMEM((1,), jnp.int32),
        ],
    )
    def k(key_hbm, out_hbm, key_v, hist_v, acc_v, cnt_s):
        cid = lax.axis_index("c")
        sid = lax.axis_index("s")

        @pl.when((cid == 0) & (sid == 0))
        def _():
            cnt_s[0] = 0

        plsc.subcore_barrier()

        @pl.when(cid == 0)
        def _():
            pltpu.sync_copy(key_hbm, key_v)
            lane = lax.iota(jnp.int32, 16)
            zeros = jnp.zeros((16,), jnp.int32)
            ones = jnp.ones((16,), jnp.int32)
            base = sid * BINS

            for j in range(BINS // 16):
                hist_v[pl.ds(j * 16, 16)] = zeros

            def body(i, _):
                kv = key_v[pl.ds(i * 16, 16)]
                binl = kv - base
                m = (binl >= 0) & (binl < BINS)
                binl = jnp.where(m, binl, 0)
                plsc.addupdate_scatter(hist_v, [binl], ones, mask=m)
                return 0

            lax.fori_loop(0, NVEC, body, 0, unroll=8)

            ps = zeros
            for kk in range(KPT // 16):
                mx = zeros
                for c in range(NCLS):
                    g = plsc.load_gather(hist_v, [kk * 16 * CSL + lane * CSL + c])
                    mx = jnp.maximum(mx, g)
                ps = ps + mx
            plsc.fetch_and_add(cnt_s.at[0], jnp.sum(ps), subcore_id=0)

        plsc.subcore_barrier()

        @pl.when((cid == 0) & (sid == 0))
        def _():
            accf = cnt_s[0].astype(jnp.float32) * (1.0 / N)
            acc_v[...] = jnp.full((16,), accf, jnp.float32)
            pltpu.sync_copy(acc_v, out_hbm)

    return k(key)


def kernel(x, y, centers):
    y3 = y.astype(jnp.int32).reshape(GRID, ROWS, 1)
    loss2d, key3 = pl.pallas_call(
        _tc_body,
        grid=(GRID,),
        in_specs=[
            pl.BlockSpec((ROWS, D), lambda i: (i, 0)),
            pl.BlockSpec((K, D), lambda i: (0, 0)),
            pl.BlockSpec((1, ROWS, 1), lambda i: (i, 0, 0)),
        ],
        out_specs=[
            pl.BlockSpec(memory_space=pltpu.SMEM),
            pl.BlockSpec((1, ROWS, 1), lambda i: (i, 0, 0)),
        ],
        out_shape=[
            jax.ShapeDtypeStruct((1, 1), jnp.float32),
            jax.ShapeDtypeStruct((GRID, ROWS, 1), jnp.int32),
        ],
    )(x, centers, y3)
    accv = _sc_hist(key3.reshape(N))
    return loss2d[0, 0], accv[0]


# speedup vs baseline: 1.0005x; 1.0005x over previous
import functools

import jax
import jax.numpy as jnp
from jax import lax
from jax.experimental import pallas as pl
from jax.experimental.pallas import tpu as pltpu
from jax.experimental.pallas import tpu_sc as plsc

N = 4096
D = 64
K = 512
NCLS = 10
CSL = 16

ROWS = 1024
GRID = N // ROWS

NTILES = 16
KPT = K // NTILES
BINS = KPT * CSL
NVEC = N // 16


def _tc_body(x_ref, c_ref, y_ref, loss_ref, key_ref):
    i = pl.program_id(0)
    xb = x_ref[...]
    cb = c_ref[...]
    xc = lax.dot_general(
        xb, cb, (((1,), (1,)), ((), ())),
        preferred_element_type=jnp.float32,
        precision=lax.Precision.HIGHEST,
    )
    xn = jnp.sum(xb * xb, axis=1)
    cn = jnp.sum(cb * cb, axis=1)
    dist = xn[:, None] + cn[None, :] - 2.0 * xc
    minv = jnp.min(dist, axis=1)
    col = lax.broadcasted_iota(jnp.int32, (ROWS, K), 1)
    yp = jnp.min(jnp.where(dist == minv[:, None], col, K), axis=1)
    key_ref[0, :, 0] = yp * CSL + y_ref[0, :, 0]

    @pl.when(i == 0)
    def _():
        loss_ref[0, 0] = 0.0

    loss_ref[0, 0] += jnp.sum(minv)


def _sc_hist(key):
    mesh = plsc.VectorSubcoreMesh(core_axis_name="c", subcore_axis_name="s")

    @functools.partial(
        pl.kernel,
        out_type=jax.ShapeDtypeStruct((16,), jnp.float32),
        mesh=mesh,
        compiler_params=pltpu.CompilerParams(needs_layout_passes=False),
        scratch_types=[
            pltpu.VMEM((N,), jnp.int32),
            pltpu.VMEM((BINS,), jnp.int32),
            pltpu.VMEM((16,), jnp.float32),
            pltpu.SMEM((1,), jnp.int32),
        ],
    )
    def k(key_hbm, out_hbm, key_v, hist_v, acc_v, cnt_s):
        cid = lax.axis_index("c")
        sid = lax.axis_index("s")

        @pl.when((cid == 0) & (sid == 0))
        def _():
            cnt_s[0] = 0

        plsc.subcore_barrier()

        @pl.when(cid == 0)
        def _():
            pltpu.sync_copy(key_hbm, key_v)
            lane = lax.iota(jnp.int32, 16)
            zeros = jnp.zeros((16,), jnp.int32)
            ones = jnp.ones((16,), jnp.int32)
            base = sid * BINS

            for j in range(BINS // 16):
                hist_v[pl.ds(j * 16, 16)] = zeros

            def body(i, _):
                kv = key_v[pl.ds(i * 16, 16)]
                binl = kv - base
                m = (binl >= 0) & (binl < BINS)
                binl = jnp.where(m, binl, 0)
                plsc.addupdate_scatter(hist_v, [binl], ones, mask=m)
                return 0

            lax.fori_loop(0, NVEC, body, 0, unroll=4)

            ps = zeros
            for kk in range(KPT // 16):
                mx = zeros
                for c in range(NCLS):
                    g = plsc.load_gather(hist_v, [kk * 16 * CSL + lane * CSL + c])
                    mx = jnp.maximum(mx, g)
                ps = ps + mx
            plsc.fetch_and_add(cnt_s.at[0], jnp.sum(ps), subcore_id=0)

        plsc.subcore_barrier()

        @pl.when((cid == 0) & (sid == 0))
        def _():
            accf = cnt_s[0].astype(jnp.float32) * (1.0 / N)
            acc_v[...] = jnp.full((16,), accf, jnp.float32)
            pltpu.sync_copy(acc_v, out_hbm)

    return k(key)


def kernel(x, y, centers):
    y3 = y.astype(jnp.int32).reshape(GRID, ROWS, 1)
    loss2d, key3 = pl.pallas_call(
        _tc_body,
        grid=(GRID,),
        in_specs=[
            pl.BlockSpec((ROWS, D), lambda i: (i, 0)),
            pl.BlockSpec((K, D), lambda i: (0, 0)),
            pl.BlockSpec((1, ROWS, 1), lambda i: (i, 0, 0)),
        ],
        out_specs=[
            pl.BlockSpec(memory_space=pltpu.SMEM),
            pl.BlockSpec((1, ROWS, 1), lambda i: (i, 0, 0)),
        ],
        out_shape=[
            jax.ShapeDtypeStruct((1, 1), jnp.float32),
            jax.ShapeDtypeStruct((GRID, ROWS, 1), jnp.int32),
        ],
    )(x, centers, y3)
    accv = _sc_hist(key3.reshape(N))
    return loss2d[0, 0], accv[0]


# speedup vs baseline: 1.0854x; 1.0849x over previous
import functools

import jax
import jax.numpy as jnp
from jax import lax
from jax.experimental import pallas as pl
from jax.experimental.pallas import tpu as pltpu
from jax.experimental.pallas import tpu_sc as plsc

N = 4096
D = 64
K = 512
NCLS = 10

ROWS = 1024
GRID = N // ROWS

NTILES = 16
KPT = K // NTILES
BINS = NCLS * KPT
NREP = 16
NVEC = N // 16


def _tc_body(x_ref, c_ref, loss_ref, yp_ref):
    i = pl.program_id(0)
    xb = x_ref[...]
    cb = c_ref[...]
    xc = lax.dot_general(
        xb, cb, (((1,), (1,)), ((), ())),
        preferred_element_type=jnp.float32,
        precision=lax.Precision.HIGHEST,
    )
    xn = jnp.sum(xb * xb, axis=1)
    cn = jnp.sum(cb * cb, axis=1)
    dist = xn[:, None] + cn[None, :] - 2.0 * xc
    minv = jnp.min(dist, axis=1)
    col = lax.broadcasted_iota(jnp.int32, (ROWS, K), 1).astype(jnp.float32)
    yp_ref[0, :, 0] = jnp.min(
        jnp.where(dist == minv[:, None], col, float(K)), axis=1
    ).astype(jnp.int32)

    @pl.when(i == 0)
    def _():
        loss_ref[0, 0] = 0.0

    loss_ref[0, 0] += jnp.sum(minv)


def _sc_hist(yp, y):
    mesh = plsc.VectorSubcoreMesh(core_axis_name="c", subcore_axis_name="s")

    @functools.partial(
        pl.kernel,
        out_type=jax.ShapeDtypeStruct((16,), jnp.float32),
        mesh=mesh,
        compiler_params=pltpu.CompilerParams(needs_layout_passes=False),
        scratch_types=[
            pltpu.VMEM((N,), jnp.int32),
            pltpu.VMEM((N,), jnp.int32),
            pltpu.VMEM((BINS,), jnp.int32),
            pltpu.VMEM((16,), jnp.float32),
            pltpu.SMEM((1,), jnp.int32),
        ],
    )
    def k(yp_hbm, y_hbm, out_hbm, yp_v, y_v, hist_v, acc_v, cnt_s):
        cid = lax.axis_index("c")
        sid = lax.axis_index("s")

        @pl.when((cid == 0) & (sid == 0))
        def _():
            cnt_s[0] = 0

        plsc.subcore_barrier()

        @pl.when(cid == 0)
        def _():
            pltpu.sync_copy(yp_hbm, yp_v)
            pltpu.sync_copy(y_hbm, y_v)
            zeros = jnp.zeros((16,), jnp.int32)
            ones = jnp.ones((16,), jnp.int32)
            lo = sid * KPT

            for j in range(BINS // 16):
                hist_v[pl.ds(j * 16, 16)] = zeros

            def body(i, _):
                ypv = yp_v[pl.ds(i * 16, 16)]
                yv = y_v[pl.ds(i * 16, 16)]
                m = (ypv >= lo) & (ypv < lo + KPT)
                binl = yv * KPT + (ypv - lo)
                binl = jnp.where(m, binl, 0)
                plsc.addupdate_scatter(hist_v, [binl], ones, mask=m)
                return 0

            lax.fori_loop(0, NVEC, body, 0, unroll=4)

            ps = zeros
            for kk in range(KPT // 16):
                mx = zeros
                for c in range(NCLS):
                    mx = jnp.maximum(mx, hist_v[pl.ds(c * KPT + kk * 16, 16)])
                ps = ps + mx
            plsc.fetch_and_add(cnt_s.at[0], jnp.sum(ps), subcore_id=0)

        plsc.subcore_barrier()

        @pl.when((cid == 0) & (sid == 0))
        def _():
            accf = cnt_s[0].astype(jnp.float32) * (1.0 / N)
            acc_v[...] = jnp.full((16,), accf, jnp.float32)
            pltpu.sync_copy(acc_v, out_hbm)

    return k(yp, y)


def kernel(x, y, centers):
    loss2d, yp = pl.pallas_call(
        _tc_body,
        grid=(GRID,),
        in_specs=[
            pl.BlockSpec((ROWS, D), lambda i: (i, 0)),
            pl.BlockSpec((K, D), lambda i: (0, 0)),
        ],
        out_specs=[
            pl.BlockSpec(memory_space=pltpu.SMEM),
            pl.BlockSpec((1, ROWS, 1), lambda i: (i, 0, 0)),
        ],
        out_shape=[
            jax.ShapeDtypeStruct((1, 1), jnp.float32),
            jax.ShapeDtypeStruct((GRID, ROWS, 1), jnp.int32),
        ],
    )(x, centers)
    accv = _sc_hist(yp.reshape(N), y.astype(jnp.int32))
    return loss2d[0, 0], accv[0]


# speedup vs baseline: 1.0870x; 1.0015x over previous
import functools

import jax
import jax.numpy as jnp
from jax import lax
from jax.experimental import pallas as pl
from jax.experimental.pallas import tpu as pltpu
from jax.experimental.pallas import tpu_sc as plsc

N = 4096
D = 64
K = 512
NCLS = 10

ROWS = 1024
GRID = N // ROWS

NTILES = 16
KPT = K // NTILES
BINS = NCLS * KPT
NVEC = N // 16


def _tc_body(x_ref, c_ref, loss_ref, yp_ref):
    i = pl.program_id(0)
    xb = x_ref[...]
    cb = c_ref[...]
    xc = lax.dot_general(
        xb, cb, (((1,), (1,)), ((), ())),
        preferred_element_type=jnp.float32,
        precision=lax.Precision.HIGHEST,
    )
    xn = jnp.sum(xb * xb, axis=1)
    cn = jnp.sum(cb * cb, axis=1)
    dist = xn[:, None] + cn[None, :] - 2.0 * xc
    minv = jnp.min(dist, axis=1)
    col = lax.broadcasted_iota(jnp.int32, (ROWS, K), 1).astype(jnp.float32)
    yp_ref[0, :, 0] = jnp.min(
        jnp.where(dist == minv[:, None], col, float(K)), axis=1
    ).astype(jnp.int32)

    @pl.when(i == 0)
    def _():
        loss_ref[0, 0] = 0.0

    loss_ref[0, 0] += jnp.sum(minv)


def _sc_hist(yp, y):
    mesh = plsc.VectorSubcoreMesh(core_axis_name="c", subcore_axis_name="s")

    @functools.partial(
        pl.kernel,
        out_type=jax.ShapeDtypeStruct((16,), jnp.float32),
        mesh=mesh,
        compiler_params=pltpu.CompilerParams(needs_layout_passes=False),
        scratch_types=[
            pltpu.VMEM((N,), jnp.int32),
            pltpu.VMEM((N,), jnp.int32),
            pltpu.VMEM((BINS,), jnp.int32),
            pltpu.VMEM((16,), jnp.float32),
            pltpu.SMEM((1,), jnp.int32),
        ],
    )
    def k(yp_hbm, y_hbm, out_hbm, yp_v, y_v, hist_v, acc_v, cnt_s):
        cid = lax.axis_index("c")
        sid = lax.axis_index("s")

        @pl.when((cid == 0) & (sid == 0))
        def _():
            cnt_s[0] = 0

        plsc.subcore_barrier()

        @pl.when(cid == 0)
        def _():
            pltpu.sync_copy(yp_hbm, yp_v)
            pltpu.sync_copy(y_hbm, y_v)
            zeros = jnp.zeros((16,), jnp.int32)
            ones = jnp.ones((16,), jnp.int32)
            lo = sid * KPT

            for j in range(BINS // 16):
                hist_v[pl.ds(j * 16, 16)] = zeros

            def body(i, _):
                ypv = yp_v[pl.ds(i * 16, 16)]
                yv = y_v[pl.ds(i * 16, 16)]
                m = (ypv >= lo) & (ypv < lo + KPT)
                binl = yv * KPT + (ypv - lo)
                binl = jnp.where(m, binl, 0)
                plsc.addupdate_scatter(hist_v, [binl], ones, mask=m)
                return 0

            lax.fori_loop(0, NVEC, body, 0, unroll=4)

            ps = zeros
            for kk in range(KPT // 16):
                mx = zeros
                for c in range(NCLS):
                    mx = jnp.maximum(mx, hist_v[pl.ds(c * KPT + kk * 16, 16)])
                ps = ps + mx
            plsc.fetch_and_add(cnt_s.at[0], jnp.sum(ps), subcore_id=0)

        plsc.subcore_barrier()

        @pl.when((cid == 0) & (sid == 0))
        def _():
            accf = cnt_s[0].astype(jnp.float32) * (1.0 / N)
            acc_v[...] = jnp.full((16,), accf, jnp.float32)
            pltpu.sync_copy(acc_v, out_hbm)

    return k(yp, y)


def kernel(x, y, centers):
    loss2d, yp = pl.pallas_call(
        _tc_body,
        grid=(GRID,),
        in_specs=[
            pl.BlockSpec((ROWS, D), lambda i: (i, 0)),
            pl.BlockSpec((K, D), lambda i: (0, 0)),
        ],
        out_specs=[
            pl.BlockSpec(memory_space=pltpu.SMEM),
            pl.BlockSpec((1, ROWS, 1), lambda i: (i, 0, 0)),
        ],
        out_shape=[
            jax.ShapeDtypeStruct((1, 1), jnp.float32),
            jax.ShapeDtypeStruct((GRID, ROWS, 1), jnp.int32),
        ],
    )(x, centers)
    accv = _sc_hist(yp.reshape(N), y.astype(jnp.int32))
    return loss2d[0, 0], accv[0]
